# Initial kernel scaffold; baseline (speedup 1.0000x reference)
import jax
import jax.numpy as jnp
from jax.experimental import pallas as pl


def kernel(edge_attr, table0, table1, table2):
    raise NotImplementedError("write your pallas kernel here")


# speedup vs baseline: 1.2121x; 1.2121x over previous
import functools

import jax
import jax.numpy as jnp
from jax import lax
from jax.experimental import pallas as pl
from jax.experimental.pallas import tpu as pltpu
from jax.experimental.pallas import tpu_sc as plsc

E = 320000
D = 128
NC, NS = 2, 16
NW = NC * NS
CHUNK = E // NW
BLK = 80
NBLK = CHUNK // BLK
GPB = BLK // 16
N0, N1, N2 = 5, 6, 2
NCOMB = N0 * N1 * N2


def _sc_body(edge_hbm, t0_hbm, t1_hbm, t2_hbm, out_hbm,
             ebuf, tb0, tb1, tb2, cflat, obuf, sem0, sem1):
    wid = lax.axis_index("s") * NC + lax.axis_index("c")
    ebase = wid * CHUNK

    pltpu.sync_copy(edge_hbm.at[pl.ds(ebase * 3, CHUNK * 3)], ebuf)
    pltpu.sync_copy(t0_hbm, tb0)
    pltpu.sync_copy(t1_hbm, tb1)
    pltpu.sync_copy(t2_hbm, tb2)

    def build_row(c, carry):
        i0 = c // (N1 * N2)
        r = c - i0 * (N1 * N2)
        i1 = r // N2
        i2 = r - i1 * N2
        for j in range(D // 16):
            s = pl.ds(j * 16, 16)
            v = tb0[i0, s] + tb1[i1, s] + tb2[i2, s]
            cflat[pl.ds(c * D + j * 16, 16)] = v
        return carry
    lax.fori_loop(0, NCOMB, build_row, 0)

    lanes = lax.iota(jnp.int32, 16)

    def fill_block(b, half):
        cvecs = []
        for g in range(GPB):
            pos = (b * BLK + g * 16) * 3
            posv = pos + lanes * 3
            a0 = plsc.load_gather(ebuf, [posv])
            a1 = plsc.load_gather(ebuf, [posv + 1])
            a2 = plsc.load_gather(ebuf, [posv + 2])
            cvecs.append((a0 * (N1 * N2) + a1 * N2 + a2) * D)
        ovecs = [half * (BLK * D) + (g * 16 + lanes) * D for g in range(GPB)]

        def jbody(j, carry):
            for g in range(GPB):
                vals = plsc.load_gather(cflat, [cvecs[g] + j])
                plsc.store_scatter(obuf, [ovecs[g] + j], vals)
            return carry
        lax.fori_loop(0, D, jbody, 0)

    def block_copy(b, half, sem):
        src = obuf.at[pl.ds(half * BLK * D, BLK * D)]
        dst = out_hbm.at[pl.ds((ebase + b * BLK) * D, BLK * D)]
        return pltpu.make_async_copy(src, dst, sem)

    sems = (sem0, sem1)

    def pair(p, carry):
        for half in (0, 1):
            b = p * 2 + half

            @pl.when(p >= 1)
            def _():
                block_copy(b, half, sems[half]).wait()

            fill_block(b, half)
            block_copy(b, half, sems[half]).start()
        return carry
    lax.fori_loop(0, NBLK // 2, pair, 0)

    b_tail = NBLK - 1
    block_copy(b_tail, 0, sem0).wait()
    fill_block(b_tail, 0)
    block_copy(b_tail, 0, sem0).start()
    block_copy(NBLK - 2, 1, sem1).wait()
    block_copy(b_tail, 0, sem0).wait()


@functools.partial(
    pl.kernel,
    out_type=jax.ShapeDtypeStruct((E * D,), jnp.float32),
    mesh=plsc.VectorSubcoreMesh(core_axis_name="c", subcore_axis_name="s"),
    compiler_params=pltpu.CompilerParams(needs_layout_passes=False),
    scratch_types=[
        pltpu.VMEM((CHUNK * 3,), jnp.int32),
        pltpu.VMEM((N0, D), jnp.float32),
        pltpu.VMEM((N1, D), jnp.float32),
        pltpu.VMEM((N2, D), jnp.float32),
        pltpu.VMEM((NCOMB * D,), jnp.float32),
        pltpu.VMEM((2 * BLK * D,), jnp.float32),
        pltpu.SemaphoreType.DMA,
        pltpu.SemaphoreType.DMA,
    ],
)
def _bond_encode_sc(edge_hbm, t0_hbm, t1_hbm, t2_hbm, out_hbm,
                    ebuf, tb0, tb1, tb2, cflat, obuf, sem0, sem1):
    _sc_body(edge_hbm, t0_hbm, t1_hbm, t2_hbm, out_hbm,
             ebuf, tb0, tb1, tb2, cflat, obuf, sem0, sem1)


def kernel(edge_attr, table0, table1, table2):
    ea = edge_attr.astype(jnp.int32).reshape(-1)
    out_flat = _bond_encode_sc(ea, table0, table1, table2)
    return out_flat.reshape(E, D)
